# Initial kernel scaffold; baseline (speedup 1.0000x reference)
import jax
import jax.numpy as jnp
from jax.experimental import pallas as pl


def kernel(pos, vel, field, index, edge_index, data_id, a, We0, We1, We2, We3, We4, be0, be1, be2, be3, be4, Wp0, Wp1, Wp2, bp0, bp1, bp2):
    raise NotImplementedError("write your pallas kernel here")


# speedup vs baseline: 22.5417x; 22.5417x over previous
import functools

import jax
import jax.numpy as jnp
from jax import lax
from jax.experimental import pallas as pl
from jax.experimental.pallas import tpu as pltpu
from jax.experimental.pallas import tpu_sc as plsc

MAX_RADIUS = 0.1
VNORM = 1.0
N = 10000
NP = 10240
NE = 640000
NC = 2
NS = 16
NW = NC * NS
EPW = 20480
NEP = NW * EPW
PAD = NEP - NE
SUB = 2048
NSUB = EPW // SUB
H = 256
F = 8
CB = 2048

_mesh = functools.partial(
    plsc.VectorSubcoreMesh,
    core_axis_name="c", subcore_axis_name="s",
    num_cores=NC, num_subcores=NS,
)


def _featurize_body(tbl_hbm, dst_hbm, src_hbm, feat_hbm, tbl_v, d_v, s_v, fst):
    c = lax.axis_index("c")
    s = lax.axis_index("s")
    wid = s * NC + c
    pltpu.sync_copy(tbl_hbm, tbl_v)
    rows = [jnp.full((16,), i, jnp.int32) for i in range(6)]
    inv_r = 1.0 / MAX_RADIUS

    def sub_body(k, carry):
        base = wid * EPW + k * SUB
        pltpu.sync_copy(dst_hbm.at[pl.ds(base, SUB)], d_v)
        pltpu.sync_copy(src_hbm.at[pl.ds(base, SUB)], s_v)

        def inner(j, carry2):
            sl = pl.ds(j * 16, 16)
            d = d_v[sl]
            sj = s_v[sl]
            pxi = plsc.load_gather(tbl_v, [rows[0], d])
            pyi = plsc.load_gather(tbl_v, [rows[1], d])
            pxj = plsc.load_gather(tbl_v, [rows[0], sj])
            pyj = plsc.load_gather(tbl_v, [rows[1], sj])
            vxi = plsc.load_gather(tbl_v, [rows[2], d])
            vyi = plsc.load_gather(tbl_v, [rows[3], d])
            vxj = plsc.load_gather(tbl_v, [rows[2], sj])
            vyj = plsc.load_gather(tbl_v, [rows[3], sj])
            exi = plsc.load_gather(tbl_v, [rows[4], d])
            eyi = plsc.load_gather(tbl_v, [rows[5], d])
            dx = (pxj - pxi) * inv_r
            dy = (pyj - pyi) * inv_r
            fst[0, sl] = dx
            fst[1, sl] = dy
            fst[2, sl] = vxi
            fst[3, sl] = vyi
            fst[4, sl] = vxj
            fst[5, sl] = vyj
            fst[6, sl] = exi
            fst[7, sl] = eyi
            return carry2

        lax.fori_loop(0, SUB // 16, inner, 0)
        pltpu.sync_copy(fst, feat_hbm.at[wid, k])
        return carry

    lax.fori_loop(0, NSUB, sub_body, 0)


_featurize = pl.kernel(
    _featurize_body,
    out_type=jax.ShapeDtypeStruct((NW, NSUB, F, SUB), jnp.float32),
    mesh=_mesh(),
    compiler_params=pltpu.CompilerParams(needs_layout_passes=False),
    scratch_types=[
        pltpu.VMEM((6, NP), jnp.float32),
        pltpu.VMEM((SUB,), jnp.int32),
        pltpu.VMEM((SUB,), jnp.int32),
        pltpu.VMEM((F, SUB), jnp.float32),
    ],
)


MD = 8


def _scatter_body(msg_hbm, dst_hbm, zero_hbm, out_hbm, m_v, m2, d_v, acc_sh):
    c = lax.axis_index("c")
    s = lax.axis_index("s")
    wid = s * NC + c

    @pl.when(s == 0)
    def _():
        pltpu.sync_copy(zero_hbm, acc_sh)

    iota = lax.iota(jnp.int32, 16)
    zv = jnp.zeros((16,), jnp.float32)
    cols = [jnp.full((16,), i, jnp.int32) for i in range(MD)]

    def zero_body(j, carry):
        lane = j * 16 + iota
        for i in range(MD):
            plsc.store_scatter(m2, [lane, cols[i]], zv)
        return carry

    lax.fori_loop(0, SUB // 16, zero_body, 0)
    plsc.subcore_barrier()

    def sub_body(k, carry):
        base = wid * EPW + k * SUB
        pltpu.sync_copy(msg_hbm.at[wid, k], m_v)
        pltpu.sync_copy(dst_hbm.at[pl.ds(base, SUB)], d_v)

        def inner(j, carry2):
            sl = pl.ds(j * 16, 16)
            lane = j * 16 + iota
            plsc.store_scatter(m2, [lane, cols[0]], m_v[0, sl])
            plsc.store_scatter(m2, [lane, cols[1]], m_v[1, sl])
            return carry2

        lax.fori_loop(0, SUB // 16, inner, 0)
        pltpu.sync_copy(m2, acc_sh.at[d_v], add=True)
        return carry

    lax.fori_loop(0, NSUB, sub_body, 0)
    plsc.subcore_barrier()

    @pl.when(s == 0)
    def _():
        pltpu.sync_copy(acc_sh, out_hbm.at[c])


_scatter = pl.kernel(
    _scatter_body,
    out_type=jax.ShapeDtypeStruct((NC, NP, MD), jnp.float32),
    mesh=_mesh(),
    compiler_params=pltpu.CompilerParams(
        use_tc_tiling_on_sc=False, needs_layout_passes=False),
    scratch_types=[
        pltpu.VMEM((2, SUB), jnp.float32),
        pltpu.VMEM((SUB, MD), jnp.float32),
        pltpu.VMEM((SUB,), jnp.int32),
        pltpu.VMEM_SHARED((NP, MD), jnp.float32),
    ],
)


def _edge_mlp_body(f_ref, w0_ref, wr_ref, b0_ref, w1_ref, b1_ref, w2_ref,
                   b2_ref, w3_ref, b3_ref, w4_ref, b4_ref, out_ref):
    f = f_ref[0, 0]
    dx = f[0:1, :]
    dy = f[1:2, :]
    r = jnp.sqrt(dx * dx + dy * dy + 1e-10)
    h = jnp.dot(w0_ref[...], f, preferred_element_type=jnp.float32)
    h = jnp.maximum(h + wr_ref[...] * r + b0_ref[...], 0.0)
    for w, b in ((w1_ref, b1_ref), (w2_ref, b2_ref), (w3_ref, b3_ref)):
        h = jnp.dot(w[...], h, preferred_element_type=jnp.float32)
        h = jnp.maximum(h + b[...], 0.0)
    out_ref[0, 0] = (
        jnp.dot(w4_ref[...], h, preferred_element_type=jnp.float32)
        + b4_ref[...])


def _full(shape):
    return pl.BlockSpec(shape, lambda i: (0,) * len(shape))


_edge_mlp = pl.pallas_call(
    _edge_mlp_body,
    grid=(NW * NSUB,),
    in_specs=[
        pl.BlockSpec((1, 1, F, SUB), lambda i: (i // NSUB, i % NSUB, 0, 0)),
        _full((H, F)), _full((H, 1)), _full((H, 1)),
        _full((H, H)), _full((H, 1)),
        _full((H, H)), _full((H, 1)),
        _full((H, H)), _full((H, 1)),
        _full((2, H)), _full((2, 1)),
    ],
    out_specs=pl.BlockSpec((1, 1, 2, SUB), lambda i: (i // NSUB, i % NSUB, 0, 0)),
    out_shape=jax.ShapeDtypeStruct((NW, NSUB, 2, SUB), jnp.float32),
)


def _phi_body(p_ref, emb_ref, dpos_ref, wp0_ref, bp0_ref, wp1_ref, bp1_ref,
              wp2_ref, bp2_ref, out_ref):
    agg = (p_ref[0] + p_ref[1])[:, 0:2]
    x = jnp.concatenate([agg, emb_ref[...], dpos_ref[...]], axis=1)
    h = jnp.dot(x, wp0_ref[...], preferred_element_type=jnp.float32)
    h = jnp.maximum(h + bp0_ref[...], 0.0)
    h = jnp.dot(h, wp1_ref[...], preferred_element_type=jnp.float32)
    h = jnp.maximum(h + bp1_ref[...], 0.0)
    out_ref[...] = (
        jnp.dot(h, wp2_ref[...], preferred_element_type=jnp.float32)
        + bp2_ref[...])


_phi = pl.pallas_call(
    _phi_body,
    grid=(NP // CB,),
    in_specs=[
        pl.BlockSpec((NC, CB, MD), lambda i: (0, i, 0)),
        pl.BlockSpec((CB, 2), lambda i: (i, 0)),
        pl.BlockSpec((CB, 2), lambda i: (i, 0)),
        _full((6, H)), _full((1, H)),
        _full((H, H)), _full((1, H)),
        _full((H, 2)), _full((1, 2)),
    ],
    out_specs=pl.BlockSpec((CB, 2), lambda i: (i, 0)),
    out_shape=jax.ShapeDtypeStruct((NP, 2), jnp.float32),
)


def kernel(pos, vel, field, index, edge_index, data_id, a,
           We0, We1, We2, We3, We4, be0, be1, be2, be3, be4,
           Wp0, Wp1, Wp2, bp0, bp1, bp2):
    f32 = jnp.float32
    dst = edge_index[0]
    src = edge_index[1]
    pad_idx = (N + (jnp.arange(PAD, dtype=jnp.int32) % (NP - N))).astype(jnp.int32)
    dst_p = jnp.concatenate([dst, pad_idx])
    src_p = jnp.concatenate([src, pad_idx])

    emb = a[data_id, index]
    d_pos = vel * (1.0 / VNORM)
    znode = jnp.zeros((NP - N,), f32)
    tbl = jnp.stack([
        jnp.concatenate([pos[:, 0], znode]),
        jnp.concatenate([pos[:, 1], znode]),
        jnp.concatenate([d_pos[:, 0], znode]),
        jnp.concatenate([d_pos[:, 1], znode]),
        jnp.concatenate([emb[:, 0], znode]),
        jnp.concatenate([emb[:, 1], znode]),
    ])

    featT = _featurize(tbl, dst_p, src_p)

    w08t = jnp.concatenate([We0[0:2], We0[3:9]], axis=0).T
    wr = We0[2:3].T
    msg = _edge_mlp(
        featT, w08t, wr, be0[:, None],
        We1.T, be1[:, None], We2.T, be2[:, None], We3.T, be3[:, None],
        We4.T, be4[:, None])

    partials = _scatter(msg, dst_p, jnp.zeros((NP, MD), f32))

    emb_p = jnp.concatenate([emb, jnp.zeros((NP - N, 2), f32)], axis=0)
    dpos_p = jnp.concatenate([d_pos, jnp.zeros((NP - N, 2), f32)], axis=0)
    out = _phi(partials, emb_p, dpos_p,
               Wp0, bp0[None, :], Wp1, bp1[None, :],
               Wp2, bp2[None, :])
    return out[:N]
